# Initial kernel scaffold; baseline (speedup 1.0000x reference)
import jax
import jax.numpy as jnp
from jax.experimental import pallas as pl


def kernel(src_feat, mp0_feat, mp1_feat, W_src, b_src, W_nbr, b_nbr, gat0_Wf, gat0_bf, gat0_Wa, gat0_ba, gat0_bias, gat1_Wf, gat1_bf, gat1_Wa, gat1_ba, gat1_bias, sem_W, sem_b, prep_W, prep_b, dnn0_W, dnn0_b, dnn0_g, dnn0_be, dnn1_W, dnn1_b, dnn1_g, dnn1_be, res_g, res_be, cls_W, cls_b, mp0_edge_index, mp1_edge_index):
    raise NotImplementedError("write your pallas kernel here")


# speedup vs baseline: 14.3593x; 14.3593x over previous
import functools

import jax
import jax.numpy as jnp
from jax import lax
from jax.experimental import pallas as pl
from jax.experimental.pallas import tpu as pltpu
from jax.experimental.pallas import tpu_sc as plsc

N = 10000
E = 320000
D = 128

RB = 1000
NBLK = N // RB

NTILE = 16
EPT = E // NTILE
CH = 80
NCHUNK = EPT // CH
FB = 128
N2 = 10112
NCHK = N2 // FB


def _prologue_body(src, m0, m1, Wsrc, bsrc, Wnbr, bnbr,
                   Wf0, bf0, Wa0, ba0, Wf1, bf1, Wa1, ba1,
                   srch_o, th0_o, th1_o, as0_o, at0_o, as1_o, at1_o):
    sh = src[...] @ Wsrc[...] + bsrc[...]
    srch_o[...] = sh
    f0 = m0[...] @ Wnbr[...] + bnbr[...]
    f1 = m1[...] @ Wnbr[...] + bnbr[...]
    t0 = f0 @ Wf0[...] + bf0[...]
    t1 = f1 @ Wf1[...] + bf1[...]
    th0_o[...] = t0
    th1_o[...] = t1
    wa0 = Wa0[...]
    wa1 = Wa1[...]
    v0 = Wf0[...] @ wa0[:D]
    v1 = Wf1[...] @ wa1[:D]
    c0 = bf0[...] @ wa0[:D] + ba0[...]
    c1 = bf1[...] @ wa1[:D] + ba1[...]
    as0_o[...] = sh @ v0 + c0
    at0_o[...] = t0 @ wa0[D:]
    as1_o[...] = sh @ v1 + c1
    at1_o[...] = t1 @ wa1[D:]


def _run_prologue(src_feat, mp0_feat, mp1_feat, W_src, b_src, W_nbr, b_nbr,
                  gat0_Wf, gat0_bf, gat0_Wa, gat0_ba,
                  gat1_Wf, gat1_bf, gat1_Wa, gat1_ba):
    row = pl.BlockSpec((RB, D), lambda i: (i, 0))
    mat = pl.BlockSpec((D, D), lambda i: (0, 0))
    vec = pl.BlockSpec((D,), lambda i: (0,))
    wa = pl.BlockSpec((2 * D, 1), lambda i: (0, 0))
    one = pl.BlockSpec((1,), lambda i: (0,))
    col = pl.BlockSpec((RB, 1), lambda i: (i, 0))
    f32 = jnp.float32
    return pl.pallas_call(
        _prologue_body,
        grid=(NBLK,),
        in_specs=[row, row, row, mat, vec, mat, vec,
                  mat, vec, wa, one, mat, vec, wa, one],
        out_specs=[row, row, row, col, col, col, col],
        out_shape=[jax.ShapeDtypeStruct((N, D), f32)] * 3 +
                  [jax.ShapeDtypeStruct((N, 1), f32)] * 4,
    )(src_feat, mp0_feat, mp1_feat, W_src, b_src, W_nbr, b_nbr,
      gat0_Wf, gat0_bf, gat0_Wa, gat0_ba, gat1_Wf, gat1_bf, gat1_Wa, gat1_ba)


def _sc_work(th_hbm, asrc_hbm, atgt_hbm, si_hbm, ti_hbm, out_hbm, den_hbm,
             asrc_v, atgt_v, si_c, ti_c, denom_v, w_c, rows_v,
             dtmp_v, cd_v, acc_sh):
    tec = lax.axis_index("s")

    def _z16(i, _):
        denom_v[pl.ds(i * 16, 16)] = jnp.zeros((16,), jnp.float32)
        return 0
    lax.fori_loop(0, N2 // 16, _z16, 0)

    def _zrow(e, _):
        for j in range(D // 16):
            rows_v[e, pl.ds(j * 16, 16)] = jnp.zeros((16,), jnp.float32)
        return 0
    lax.fori_loop(0, FB, _zrow, 0)

    nch = jnp.where(tec == NTILE - 1, 4, 5)

    def _zacc(k, _):
        c = tec + NTILE * k
        pltpu.sync_copy(rows_v, acc_sh.at[pl.ds(c * FB, FB)])
        return 0
    lax.fori_loop(0, nch, _zacc, 0)

    pltpu.sync_copy(asrc_hbm, asrc_v)
    pltpu.sync_copy(atgt_hbm, atgt_v)

    plsc.subcore_barrier()

    def _chunk(g, _):
        pltpu.sync_copy(si_hbm.at[tec, g], si_c)
        pltpu.sync_copy(ti_hbm.at[tec, g], ti_c)
        for b in range(CH // 16):
            sl = pl.ds(b * 16, 16)
            s16 = si_c[sl]
            t16 = ti_c[sl]
            av = plsc.load_gather(asrc_v, [s16])
            bv = plsc.load_gather(atgt_v, [t16])
            x = av + bv
            ex2 = jnp.exp(x + x)
            w = jnp.exp(1.0 - 2.0 / (ex2 + 1.0))
            w_c[sl] = w
            plsc.addupdate_scatter(denom_v, [s16], w)
        pltpu.sync_copy(th_hbm.at[ti_c], rows_v.at[pl.ds(0, CH)])

        def _scale(b, _):
            w16 = w_c[pl.ds(b * 16, 16)]
            for l in range(16):
                e = b * 16 + l
                ws = w16[l]
                for j in range(D // 16):
                    sj = pl.ds(j * 16, 16)
                    rows_v[e, sj] = rows_v[e, sj] * ws
            return 0
        lax.fori_loop(0, CH // 16, _scale, 0)

        pltpu.sync_copy(rows_v.at[pl.ds(0, CH)], acc_sh.at[si_c], add=True)
        return 0
    lax.fori_loop(0, NCHUNK, _chunk, 0)

    plsc.subcore_barrier()

    pltpu.sync_copy(denom_v, den_hbm.at[pl.ds(tec * N2, N2)])
    plsc.subcore_barrier()

    def _fin(k, _):
        c = tec + NTILE * k
        row0 = c * FB
        pltpu.sync_copy(acc_sh.at[pl.ds(row0, FB)], rows_v)
        dtot = [None] * (FB // 16)
        for p in range(NTILE):
            pltpu.sync_copy(den_hbm.at[pl.ds(p * N2 + row0, FB)], dtmp_v)
            for b in range(FB // 16):
                v = dtmp_v[pl.ds(b * 16, 16)]
                dtot[b] = v if p == 0 else dtot[b] + v
        for b in range(FB // 16):
            good = dtot[b] > 0.0
            cd_v[pl.ds(b * 16, 16)] = jnp.where(
                good, 1.0 / jnp.where(good, dtot[b], 1.0), 0.0)

        def _dr(b, _):
            r16 = cd_v[pl.ds(b * 16, 16)]
            for l in range(16):
                e = b * 16 + l
                rs = r16[l]
                for j in range(D // 16):
                    sj = pl.ds(j * 16, 16)
                    rows_v[e, sj] = rows_v[e, sj] * rs
            return 0
        lax.fori_loop(0, FB // 16, _dr, 0)

        @pl.when(c == NCHK - 1)
        def _():
            pltpu.sync_copy(rows_v.at[pl.ds(0, N - FB * (NCHK - 1))],
                            out_hbm.at[pl.ds(row0, N - FB * (NCHK - 1))])

        @pl.when(c != NCHK - 1)
        def _():
            pltpu.sync_copy(rows_v, out_hbm.at[pl.ds(row0, FB)])
        return 0
    lax.fori_loop(0, nch, _fin, 0)


def _sc_body(th0, th1, as0, at0, as1, at1, si0, ti0, si1, ti1,
             h0o, h1o, den0_o, den1_o,
             asrc_v, atgt_v, si_c, ti_c, denom_v, w_c, rows_v,
             dtmp_v, cd_v, acc_sh):
    c = lax.axis_index("c")

    @pl.when(c == 0)
    def _():
        _sc_work(th0, as0, at0, si0, ti0, h0o, den0_o,
                 asrc_v, atgt_v, si_c, ti_c, denom_v, w_c, rows_v,
                 dtmp_v, cd_v, acc_sh)

    @pl.when(c == 1)
    def _():
        _sc_work(th1, as1, at1, si1, ti1, h1o, den1_o,
                 asrc_v, atgt_v, si_c, ti_c, denom_v, w_c, rows_v,
                 dtmp_v, cd_v, acc_sh)


def _run_sc(th0, th1, as0, at0, as1, at1, si0, ti0, si1, ti1):
    f32 = jnp.float32
    mesh = plsc.VectorSubcoreMesh(core_axis_name="c", subcore_axis_name="s")
    call = pl.kernel(
        _sc_body,
        compiler_params=pltpu.CompilerParams(needs_layout_passes=False,
                                             use_tc_tiling_on_sc=False),
        out_type=[jax.ShapeDtypeStruct((N, D), f32),
                  jax.ShapeDtypeStruct((N, D), f32),
                  jax.ShapeDtypeStruct((NTILE * N2,), f32),
                  jax.ShapeDtypeStruct((NTILE * N2,), f32)],
        mesh=mesh,
        scratch_types=[
            pltpu.VMEM((N,), f32),
            pltpu.VMEM((N,), f32),
            pltpu.VMEM((CH,), jnp.int32),
            pltpu.VMEM((CH,), jnp.int32),
            pltpu.VMEM((N2,), f32),
            pltpu.VMEM((CH,), f32),
            pltpu.VMEM((FB, D), f32),
            pltpu.VMEM((FB,), f32),
            pltpu.VMEM((FB,), f32),
            pltpu.VMEM_SHARED((N2, D), f32),
        ],
    )
    h0n, h1n, _, _ = call(th0, th1, as0, at0, as1, at1, si0, ti0, si1, ti1)
    return h0n, h1n


def _ln(x, g, b):
    m = jnp.mean(x, axis=-1, keepdims=True)
    v = jnp.mean((x - m) * (x - m), axis=-1, keepdims=True)
    return (x - m) / jnp.sqrt(v + 1e-5) * g + b


def _epilogue_body(srch, h0n, h1n, g0b, g1b, semW, semb, prepW, prepb,
                   d0W, d0b, d0g, d0be, d1W, d1b, d1g, d1be,
                   rg, rbe, clsW, clsb, out_o):
    s = srch[...]
    h0 = h0n[...] + g0b[...]
    h1 = h1n[...] + g1b[...]
    sw = semW[...]
    sb = semb[...]
    a0 = s @ sw + sb
    a1 = h0 @ sw + sb
    a2 = h1 @ sw + sb
    att = jnp.concatenate([a0, a1, a2], axis=1)
    att = jnp.where(att > 0, att, 0.01 * att)
    att = att - jnp.max(att, axis=1, keepdims=True)
    ea = jnp.exp(att)
    p = ea / jnp.sum(ea, axis=1, keepdims=True)
    hp = p[:, 0:1] * s + p[:, 1:2] * h0 + p[:, 2:3] * h1
    h = hp @ prepW[...] + prepb[...]
    hs = h
    h = _ln(jnp.tanh(h @ d0W[...] + d0b[...]), d0g[...], d0be[...])
    h = _ln(jnp.tanh(h @ d1W[...] + d1b[...]), d1g[...], d1be[...])
    h = _ln(jnp.tanh(hs + h), rg[...], rbe[...])
    z = h @ clsW[...] + clsb[...]
    out_o[...] = 1.0 / (1.0 + jnp.exp(-z))


def _run_epilogue(srch, h0n, h1n, gat0_bias, gat1_bias, sem_W, sem_b,
                  prep_W, prep_b, dnn0_W, dnn0_b, dnn0_g, dnn0_be,
                  dnn1_W, dnn1_b, dnn1_g, dnn1_be, res_g, res_be,
                  cls_W, cls_b):
    row = pl.BlockSpec((RB, D), lambda i: (i, 0))
    mat = pl.BlockSpec((D, D), lambda i: (0, 0))
    vec = pl.BlockSpec((D,), lambda i: (0,))
    cvec = pl.BlockSpec((D, 1), lambda i: (0, 0))
    one = pl.BlockSpec((1,), lambda i: (0,))
    col = pl.BlockSpec((RB, 1), lambda i: (i, 0))
    return pl.pallas_call(
        _epilogue_body,
        grid=(NBLK,),
        in_specs=[row, row, row, vec, vec, cvec, one, mat, vec,
                  mat, vec, vec, vec, mat, vec, vec, vec,
                  vec, vec, cvec, one],
        out_specs=col,
        out_shape=jax.ShapeDtypeStruct((N, 1), jnp.float32),
    )(srch, h0n, h1n, gat0_bias, gat1_bias, sem_W, sem_b, prep_W, prep_b,
      dnn0_W, dnn0_b, dnn0_g, dnn0_be, dnn1_W, dnn1_b, dnn1_g, dnn1_be,
      res_g, res_be, cls_W, cls_b)


def kernel(src_feat, mp0_feat, mp1_feat, W_src, b_src, W_nbr, b_nbr,
           gat0_Wf, gat0_bf, gat0_Wa, gat0_ba, gat0_bias,
           gat1_Wf, gat1_bf, gat1_Wa, gat1_ba, gat1_bias,
           sem_W, sem_b, prep_W, prep_b,
           dnn0_W, dnn0_b, dnn0_g, dnn0_be,
           dnn1_W, dnn1_b, dnn1_g, dnn1_be,
           res_g, res_be, cls_W, cls_b,
           mp0_edge_index, mp1_edge_index):
    srch, th0, th1, as0, at0, as1, at1 = _run_prologue(
        src_feat, mp0_feat, mp1_feat, W_src, b_src, W_nbr, b_nbr,
        gat0_Wf, gat0_bf, gat0_Wa, gat0_ba, gat1_Wf, gat1_bf, gat1_Wa,
        gat1_ba)

    si0 = mp0_edge_index[0].reshape(NTILE, NCHUNK, CH)
    ti0 = mp0_edge_index[1].reshape(NTILE, NCHUNK, CH)
    si1 = mp1_edge_index[0].reshape(NTILE, NCHUNK, CH)
    ti1 = mp1_edge_index[1].reshape(NTILE, NCHUNK, CH)

    h0n, h1n = _run_sc(th0, th1,
                       as0.reshape(N), at0.reshape(N),
                       as1.reshape(N), at1.reshape(N),
                       si0, ti0, si1, ti1)

    out = _run_epilogue(srch, h0n, h1n, gat0_bias, gat1_bias, sem_W, sem_b,
                        prep_W, prep_b, dnn0_W, dnn0_b, dnn0_g, dnn0_be,
                        dnn1_W, dnn1_b, dnn1_g, dnn1_be, res_g, res_be,
                        cls_W, cls_b)
    return out.reshape(N)


# speedup vs baseline: 21.2888x; 1.4826x over previous
import functools

import jax
import jax.numpy as jnp
from jax import lax
from jax.experimental import pallas as pl
from jax.experimental.pallas import tpu as pltpu
from jax.experimental.pallas import tpu_sc as plsc

N = 10000
E = 320000
D = 128

RB = 1000
NBLK = N // RB

NTILE = 16
EPT = E // NTILE
CH = 80
NCHUNK = EPT // CH
FB = 80
NCHK = N // FB


def _prologue_body(src, m0, m1, Wsrc, bsrc, Wnbr, bnbr,
                   Wf0, bf0, Wa0, ba0, Wf1, bf1, Wa1, ba1,
                   srch_o, th0_o, th1_o, as0_o, at0_o, as1_o, at1_o):
    sh = src[...] @ Wsrc[...] + bsrc[...]
    srch_o[...] = sh
    f0 = m0[...] @ Wnbr[...] + bnbr[...]
    f1 = m1[...] @ Wnbr[...] + bnbr[...]
    t0 = f0 @ Wf0[...] + bf0[...]
    t1 = f1 @ Wf1[...] + bf1[...]
    th0_o[...] = t0
    th1_o[...] = t1
    wa0 = Wa0[...]
    wa1 = Wa1[...]
    v0 = Wf0[...] @ wa0[:D]
    v1 = Wf1[...] @ wa1[:D]
    c0 = bf0[...] @ wa0[:D] + ba0[...]
    c1 = bf1[...] @ wa1[:D] + ba1[...]
    as0_o[...] = sh @ v0 + c0
    at0_o[...] = t0 @ wa0[D:]
    as1_o[...] = sh @ v1 + c1
    at1_o[...] = t1 @ wa1[D:]


def _run_prologue(src_feat, mp0_feat, mp1_feat, W_src, b_src, W_nbr, b_nbr,
                  gat0_Wf, gat0_bf, gat0_Wa, gat0_ba,
                  gat1_Wf, gat1_bf, gat1_Wa, gat1_ba):
    row = pl.BlockSpec((RB, D), lambda i: (i, 0))
    mat = pl.BlockSpec((D, D), lambda i: (0, 0))
    vec = pl.BlockSpec((D,), lambda i: (0,))
    wa = pl.BlockSpec((2 * D, 1), lambda i: (0, 0))
    one = pl.BlockSpec((1,), lambda i: (0,))
    col = pl.BlockSpec((RB, 1), lambda i: (i, 0))
    f32 = jnp.float32
    return pl.pallas_call(
        _prologue_body,
        grid=(NBLK,),
        in_specs=[row, row, row, mat, vec, mat, vec,
                  mat, vec, wa, one, mat, vec, wa, one],
        out_specs=[row, row, row, col, col, col, col],
        out_shape=[jax.ShapeDtypeStruct((N, D), f32)] * 3 +
                  [jax.ShapeDtypeStruct((N, 1), f32)] * 4,
    )(src_feat, mp0_feat, mp1_feat, W_src, b_src, W_nbr, b_nbr,
      gat0_Wf, gat0_bf, gat0_Wa, gat0_ba, gat1_Wf, gat1_bf, gat1_Wa, gat1_ba)


def _sc_work(th_hbm, asrc_hbm, atgt_hbm, e_hbm, out_hbm,
             asrc_v, atgt_v, si_b, ti_b, w_b, rows_b,
             dtmp_v, cd_v, acc_sh, den_sp,
             sem_i, sem_g, sem_s, sem_d):
    tec = lax.axis_index("s")
    ebase = tec * EPT

    nch = jnp.where(tec <= (NCHK - 1) % NTILE, NCHK // NTILE + 1,
                    NCHK // NTILE)

    def _zrow(e, _):
        for j in range(D // 16):
            rows_b[0][e, pl.ds(j * 16, 16)] = jnp.zeros((16,), jnp.float32)
        return 0
    lax.fori_loop(0, FB, _zrow, 0)
    for b in range(FB // 16):
        dtmp_v[pl.ds(b * 16, 16)] = jnp.zeros((16,), jnp.float32)

    def _zacc(k, _):
        c = tec + NTILE * k
        pltpu.sync_copy(rows_b[0], acc_sh.at[pl.ds(c * FB, FB)])
        pltpu.sync_copy(dtmp_v, den_sp.at[pl.ds(c * FB, FB)])
        return 0
    lax.fori_loop(0, nch, _zacc, 0)

    pltpu.sync_copy(asrc_hbm, asrc_v)
    pltpu.sync_copy(atgt_hbm, atgt_v)

    plsc.subcore_barrier()

    def _issue_idx(g, q):
        off = ebase + g * CH
        pltpu.async_copy(e_hbm.at[0, pl.ds(off, CH)], si_b[q], sem_i[q])
        pltpu.async_copy(e_hbm.at[1, pl.ds(off, CH)], ti_b[q], sem_i[q])

    def _wait_idx(g, q):
        off = ebase + g * CH
        pltpu.make_async_copy(e_hbm.at[0, pl.ds(off, CH)], si_b[q],
                              sem_i[q]).wait()
        pltpu.make_async_copy(e_hbm.at[1, pl.ds(off, CH)], ti_b[q],
                              sem_i[q]).wait()

    def _wait_scat(q):
        pltpu.make_async_copy(rows_b[q], acc_sh.at[si_b[q]], sem_s[q]).wait()
        pltpu.make_async_copy(w_b[q], den_sp.at[si_b[q]], sem_d[q]).wait()

    def _chunk(g, m, q, first):
        _wait_idx(g, q)
        gat = pltpu.async_copy(th_hbm.at[ti_b[q]], rows_b[q], sem_g[q])
        for b in range(CH // 16):
            sl = pl.ds(b * 16, 16)
            s16 = si_b[q][sl]
            t16 = ti_b[q][sl]
            x = (plsc.load_gather(asrc_v, [s16]) +
                 plsc.load_gather(atgt_v, [t16]))
            ex2 = jnp.exp(x + x)
            w_b[q][sl] = jnp.exp(1.0 - 2.0 / (ex2 + 1.0))
        gat.wait()

        def _scale(b, _):
            w16 = w_b[q][pl.ds(b * 16, 16)]
            for l in range(16):
                e = b * 16 + l
                ws = w16[l]
                for j in range(D // 16):
                    sj = pl.ds(j * 16, 16)
                    rows_b[q][e, sj] = rows_b[q][e, sj] * ws
            return 0
        lax.fori_loop(0, CH // 16, _scale, 0)

        pltpu.async_copy(rows_b[q], acc_sh.at[si_b[q]], sem_s[q], add=True)
        pltpu.async_copy(w_b[q], den_sp.at[si_b[q]], sem_d[q], add=True)

        if first:
            if q == 0:
                _issue_idx(g + 1, 1)
            else:
                _wait_scat(0)
                _issue_idx(g + 1, 0)
        else:
            if q == 0:
                _wait_scat(1)
                _issue_idx(g + 1, 1)
            else:
                @pl.when(m < NCHUNK // 2 - 1)
                def _():
                    _wait_scat(0)
                    _issue_idx(g + 1, 0)

    _issue_idx(0, 0)

    def _pair(m, _):
        @pl.when(m == 0)
        def _():
            _chunk(2 * m, m, 0, True)
            _chunk(2 * m + 1, m, 1, True)

        @pl.when(m > 0)
        def _():
            _chunk(2 * m, m, 0, False)
            _chunk(2 * m + 1, m, 1, False)
        return 0
    lax.fori_loop(0, NCHUNK // 2, _pair, 0)

    _wait_scat(0)
    _wait_scat(1)

    plsc.subcore_barrier()

    def _fin(k, _):
        c = tec + NTILE * k
        row0 = c * FB
        pltpu.sync_copy(acc_sh.at[pl.ds(row0, FB)], rows_b[0])
        pltpu.sync_copy(den_sp.at[pl.ds(row0, FB)], dtmp_v)
        for b in range(FB // 16):
            sl = pl.ds(b * 16, 16)
            dtot = dtmp_v[sl]
            good = dtot > 0.0
            cd_v[sl] = jnp.where(good, 1.0 / jnp.where(good, dtot, 1.0), 0.0)

        def _dr(b, _):
            r16 = cd_v[pl.ds(b * 16, 16)]
            for l in range(16):
                e = b * 16 + l
                rs = r16[l]
                for j in range(D // 16):
                    sj = pl.ds(j * 16, 16)
                    rows_b[0][e, sj] = rows_b[0][e, sj] * rs
            return 0
        lax.fori_loop(0, FB // 16, _dr, 0)
        pltpu.sync_copy(rows_b[0], out_hbm.at[pl.ds(row0, FB)])
        return 0
    lax.fori_loop(0, nch, _fin, 0)


def _sc_body(th0, th1, as0, at0, as1, at1, e0, e1, h0o, h1o,
             asrc_v, atgt_v, si0_v, ti0_v, si1_v, ti1_v, w0_v, w1_v,
             rows0_v, rows1_v, dtmp_v, cd_v, acc_sh, den_sp,
             si0s, si1s, sg0, sg1, ss0, ss1, sd0, sd1):
    c = lax.axis_index("c")
    si_b = (si0_v, si1_v)
    ti_b = (ti0_v, ti1_v)
    w_b = (w0_v, w1_v)
    rows_b = (rows0_v, rows1_v)
    sem_i = (si0s, si1s)
    sem_g = (sg0, sg1)
    sem_s = (ss0, ss1)
    sem_d = (sd0, sd1)

    @pl.when(c == 0)
    def _():
        _sc_work(th0, as0, at0, e0, h0o,
                 asrc_v, atgt_v, si_b, ti_b, w_b, rows_b,
                 dtmp_v, cd_v, acc_sh, den_sp, sem_i, sem_g, sem_s, sem_d)

    @pl.when(c == 1)
    def _():
        _sc_work(th1, as1, at1, e1, h1o,
                 asrc_v, atgt_v, si_b, ti_b, w_b, rows_b,
                 dtmp_v, cd_v, acc_sh, den_sp, sem_i, sem_g, sem_s, sem_d)


def _run_sc(th0, th1, as0, at0, as1, at1, e0, e1):
    f32 = jnp.float32
    i32 = jnp.int32
    mesh = plsc.VectorSubcoreMesh(core_axis_name="c", subcore_axis_name="s")
    call = pl.kernel(
        _sc_body,
        compiler_params=pltpu.CompilerParams(needs_layout_passes=False,
                                             use_tc_tiling_on_sc=False),
        out_type=[jax.ShapeDtypeStruct((N, D), f32),
                  jax.ShapeDtypeStruct((N, D), f32)],
        mesh=mesh,
        scratch_types=[
            pltpu.VMEM((N,), f32),
            pltpu.VMEM((N,), f32),
            pltpu.VMEM((CH,), i32),
            pltpu.VMEM((CH,), i32),
            pltpu.VMEM((CH,), i32),
            pltpu.VMEM((CH,), i32),
            pltpu.VMEM((CH,), f32),
            pltpu.VMEM((CH,), f32),
            pltpu.VMEM((CH, D), f32),
            pltpu.VMEM((CH, D), f32),
            pltpu.VMEM((FB,), f32),
            pltpu.VMEM((FB,), f32),
            pltpu.VMEM_SHARED((N, D), f32),
            pltpu.VMEM_SHARED((N,), f32),
            pltpu.SemaphoreType.DMA,
            pltpu.SemaphoreType.DMA,
            pltpu.SemaphoreType.DMA,
            pltpu.SemaphoreType.DMA,
            pltpu.SemaphoreType.DMA,
            pltpu.SemaphoreType.DMA,
            pltpu.SemaphoreType.DMA,
            pltpu.SemaphoreType.DMA,
        ],
    )
    return call(th0, th1, as0, at0, as1, at1, e0, e1)


def _ln(x, g, b):
    m = jnp.mean(x, axis=-1, keepdims=True)
    v = jnp.mean((x - m) * (x - m), axis=-1, keepdims=True)
    return (x - m) / jnp.sqrt(v + 1e-5) * g + b


def _epilogue_body(srch, h0n, h1n, g0b, g1b, semW, semb, prepW, prepb,
                   d0W, d0b, d0g, d0be, d1W, d1b, d1g, d1be,
                   rg, rbe, clsW, clsb, out_o):
    s = srch[...]
    h0 = h0n[...] + g0b[...]
    h1 = h1n[...] + g1b[...]
    sw = semW[...]
    sb = semb[...]
    a0 = s @ sw + sb
    a1 = h0 @ sw + sb
    a2 = h1 @ sw + sb
    att = jnp.concatenate([a0, a1, a2], axis=1)
    att = jnp.where(att > 0, att, 0.01 * att)
    att = att - jnp.max(att, axis=1, keepdims=True)
    ea = jnp.exp(att)
    p = ea / jnp.sum(ea, axis=1, keepdims=True)
    hp = p[:, 0:1] * s + p[:, 1:2] * h0 + p[:, 2:3] * h1
    h = hp @ prepW[...] + prepb[...]
    hs = h
    h = _ln(jnp.tanh(h @ d0W[...] + d0b[...]), d0g[...], d0be[...])
    h = _ln(jnp.tanh(h @ d1W[...] + d1b[...]), d1g[...], d1be[...])
    h = _ln(jnp.tanh(hs + h), rg[...], rbe[...])
    z = h @ clsW[...] + clsb[...]
    out_o[...] = 1.0 / (1.0 + jnp.exp(-z))


def _run_epilogue(srch, h0n, h1n, gat0_bias, gat1_bias, sem_W, sem_b,
                  prep_W, prep_b, dnn0_W, dnn0_b, dnn0_g, dnn0_be,
                  dnn1_W, dnn1_b, dnn1_g, dnn1_be, res_g, res_be,
                  cls_W, cls_b):
    row = pl.BlockSpec((RB, D), lambda i: (i, 0))
    mat = pl.BlockSpec((D, D), lambda i: (0, 0))
    vec = pl.BlockSpec((D,), lambda i: (0,))
    cvec = pl.BlockSpec((D, 1), lambda i: (0, 0))
    one = pl.BlockSpec((1,), lambda i: (0,))
    col = pl.BlockSpec((RB, 1), lambda i: (i, 0))
    return pl.pallas_call(
        _epilogue_body,
        grid=(NBLK,),
        in_specs=[row, row, row, vec, vec, cvec, one, mat, vec,
                  mat, vec, vec, vec, mat, vec, vec, vec,
                  vec, vec, cvec, one],
        out_specs=col,
        out_shape=jax.ShapeDtypeStruct((N, 1), jnp.float32),
    )(srch, h0n, h1n, gat0_bias, gat1_bias, sem_W, sem_b, prep_W, prep_b,
      dnn0_W, dnn0_b, dnn0_g, dnn0_be, dnn1_W, dnn1_b, dnn1_g, dnn1_be,
      res_g, res_be, cls_W, cls_b)


def kernel(src_feat, mp0_feat, mp1_feat, W_src, b_src, W_nbr, b_nbr,
           gat0_Wf, gat0_bf, gat0_Wa, gat0_ba, gat0_bias,
           gat1_Wf, gat1_bf, gat1_Wa, gat1_ba, gat1_bias,
           sem_W, sem_b, prep_W, prep_b,
           dnn0_W, dnn0_b, dnn0_g, dnn0_be,
           dnn1_W, dnn1_b, dnn1_g, dnn1_be,
           res_g, res_be, cls_W, cls_b,
           mp0_edge_index, mp1_edge_index):
    srch, th0, th1, as0, at0, as1, at1 = _run_prologue(
        src_feat, mp0_feat, mp1_feat, W_src, b_src, W_nbr, b_nbr,
        gat0_Wf, gat0_bf, gat0_Wa, gat0_ba, gat1_Wf, gat1_bf, gat1_Wa,
        gat1_ba)

    h0n, h1n = _run_sc(th0, th1,
                       as0.reshape(N), at0.reshape(N),
                       as1.reshape(N), at1.reshape(N),
                       mp0_edge_index, mp1_edge_index)

    out = _run_epilogue(srch, h0n, h1n, gat0_bias, gat1_bias, sem_W, sem_b,
                        prep_W, prep_b, dnn0_W, dnn0_b, dnn0_g, dnn0_be,
                        dnn1_W, dnn1_b, dnn1_g, dnn1_be, res_g, res_be,
                        cls_W, cls_b)
    return out.reshape(N)


# speedup vs baseline: 25.6732x; 1.2059x over previous
import functools

import jax
import jax.numpy as jnp
from jax import lax
from jax.experimental import pallas as pl
from jax.experimental.pallas import tpu as pltpu
from jax.experimental.pallas import tpu_sc as plsc

N = 10000
E = 320000
D = 128

RB = 1000
NBLK = N // RB

NTILE = 16
EPT = E // NTILE
CH = 80
NCHUNK = EPT // CH
FB = 80
NCHK = N // FB


def _prologue_body(src, m0, m1, Wsrc, bsrc, Wnbr, bnbr,
                   Wf0, bf0, Wa0, ba0, Wf1, bf1, Wa1, ba1,
                   srch_o, th0_o, th1_o, as0_o, at0_o, as1_o, at1_o):
    sh = src[...] @ Wsrc[...] + bsrc[...]
    srch_o[...] = sh
    f0 = m0[...] @ Wnbr[...] + bnbr[...]
    f1 = m1[...] @ Wnbr[...] + bnbr[...]
    t0 = f0 @ Wf0[...] + bf0[...]
    t1 = f1 @ Wf1[...] + bf1[...]
    th0_o[...] = t0
    th1_o[...] = t1
    wa0 = Wa0[...]
    wa1 = Wa1[...]
    v0 = Wf0[...] @ wa0[:D]
    v1 = Wf1[...] @ wa1[:D]
    c0 = bf0[...] @ wa0[:D] + ba0[...]
    c1 = bf1[...] @ wa1[:D] + ba1[...]
    as0_o[...] = sh @ v0 + c0
    at0_o[...] = t0 @ wa0[D:]
    as1_o[...] = sh @ v1 + c1
    at1_o[...] = t1 @ wa1[D:]


def _run_prologue(src_feat, mp0_feat, mp1_feat, W_src, b_src, W_nbr, b_nbr,
                  gat0_Wf, gat0_bf, gat0_Wa, gat0_ba,
                  gat1_Wf, gat1_bf, gat1_Wa, gat1_ba):
    row = pl.BlockSpec((RB, D), lambda i: (i, 0))
    mat = pl.BlockSpec((D, D), lambda i: (0, 0))
    vec = pl.BlockSpec((D,), lambda i: (0,))
    wa = pl.BlockSpec((2 * D, 1), lambda i: (0, 0))
    one = pl.BlockSpec((1,), lambda i: (0,))
    col = pl.BlockSpec((RB, 1), lambda i: (i, 0))
    f32 = jnp.float32
    return pl.pallas_call(
        _prologue_body,
        grid=(NBLK,),
        in_specs=[row, row, row, mat, vec, mat, vec,
                  mat, vec, wa, one, mat, vec, wa, one],
        out_specs=[row, row, row, col, col, col, col],
        out_shape=[jax.ShapeDtypeStruct((N, D), f32)] * 3 +
                  [jax.ShapeDtypeStruct((N, 1), f32)] * 4,
    )(src_feat, mp0_feat, mp1_feat, W_src, b_src, W_nbr, b_nbr,
      gat0_Wf, gat0_bf, gat0_Wa, gat0_ba, gat1_Wf, gat1_bf, gat1_Wa, gat1_ba)


def _sc_work(th_hbm, asrc_hbm, atgt_hbm, e_hbm, out_hbm,
             asrc_v, atgt_v, si_b, ti_b, w_b, rows_b,
             dtmp_v, cd_v, acc_sh, den_sp,
             sem_i, sem_g, sem_s, sem_d):
    tec = lax.axis_index("s")
    ebase = tec * EPT

    nch = jnp.where(tec <= (NCHK - 1) % NTILE, NCHK // NTILE + 1,
                    NCHK // NTILE)

    def _zrow(e, _):
        for j in range(D // 16):
            rows_b[0][e, pl.ds(j * 16, 16)] = jnp.zeros((16,), jnp.float32)
        return 0
    lax.fori_loop(0, FB, _zrow, 0)
    for b in range(FB // 16):
        dtmp_v[pl.ds(b * 16, 16)] = jnp.zeros((16,), jnp.float32)

    def _zacc(k, _):
        c = tec + NTILE * k
        pltpu.sync_copy(rows_b[0], acc_sh.at[pl.ds(c * FB, FB)])
        pltpu.sync_copy(dtmp_v, den_sp.at[pl.ds(c * FB, FB)])
        return 0
    lax.fori_loop(0, nch, _zacc, 0)

    pltpu.sync_copy(asrc_hbm, asrc_v)
    pltpu.sync_copy(atgt_hbm, atgt_v)

    plsc.subcore_barrier()

    def _issue_idx(g, q):
        off = ebase + g * CH
        pltpu.async_copy(e_hbm.at[0, pl.ds(off, CH)], si_b[q], sem_i[q])
        pltpu.async_copy(e_hbm.at[1, pl.ds(off, CH)], ti_b[q], sem_i[q])

    def _wait_idx(g, q):
        off = ebase + g * CH
        pltpu.make_async_copy(e_hbm.at[0, pl.ds(off, CH)], si_b[q],
                              sem_i[q]).wait()
        pltpu.make_async_copy(e_hbm.at[1, pl.ds(off, CH)], ti_b[q],
                              sem_i[q]).wait()

    def _wait_scat(q):
        pltpu.make_async_copy(rows_b[q], acc_sh.at[si_b[q]], sem_s[q]).wait()
        pltpu.make_async_copy(w_b[q], den_sp.at[si_b[q]], sem_d[q]).wait()

    def _chunk(g, m, q, last):
        if q == 0:
            @pl.when(m >= 1)
            def _():
                _wait_scat(1)
        else:
            _wait_scat(0)
        if not last:
            _issue_idx(g + 1, 1 - q)
        for b in range(CH // 16):
            sl = pl.ds(b * 16, 16)
            s16 = si_b[q][sl]
            t16 = ti_b[q][sl]
            x = (plsc.load_gather(asrc_v, [s16]) +
                 plsc.load_gather(atgt_v, [t16]))
            ex2 = jnp.exp(x + x)
            w_b[q][sl] = jnp.exp(1.0 - 2.0 / (ex2 + 1.0))
        pltpu.make_async_copy(th_hbm.at[ti_b[q]], rows_b[q],
                              sem_g[q]).wait()

        def _scale(b, _):
            w16 = w_b[q][pl.ds(b * 16, 16)]
            for l in range(16):
                e = b * 16 + l
                ws = w16[l]
                for j in range(D // 16):
                    sj = pl.ds(j * 16, 16)
                    rows_b[q][e, sj] = rows_b[q][e, sj] * ws
            return 0
        lax.fori_loop(0, CH // 16, _scale, 0)

        pltpu.async_copy(rows_b[q], acc_sh.at[si_b[q]], sem_s[q], add=True)
        pltpu.async_copy(w_b[q], den_sp.at[si_b[q]], sem_d[q], add=True)

        if not last:
            _wait_idx(g + 1, 1 - q)
            pltpu.async_copy(th_hbm.at[ti_b[1 - q]], rows_b[1 - q],
                             sem_g[1 - q])

    _issue_idx(0, 0)
    _wait_idx(0, 0)
    pltpu.async_copy(th_hbm.at[ti_b[0]], rows_b[0], sem_g[0])

    def _pair(m, _):
        _chunk(2 * m, m, 0, False)

        @pl.when(m == NCHUNK // 2 - 1)
        def _():
            _chunk(2 * m + 1, m, 1, True)

        @pl.when(m < NCHUNK // 2 - 1)
        def _():
            _chunk(2 * m + 1, m, 1, False)
        return 0
    lax.fori_loop(0, NCHUNK // 2, _pair, 0)

    _wait_scat(1)

    plsc.subcore_barrier()

    def _fin(k, _):
        c = tec + NTILE * k
        row0 = c * FB
        pltpu.sync_copy(acc_sh.at[pl.ds(row0, FB)], rows_b[0])
        pltpu.sync_copy(den_sp.at[pl.ds(row0, FB)], dtmp_v)
        for b in range(FB // 16):
            sl = pl.ds(b * 16, 16)
            dtot = dtmp_v[sl]
            good = dtot > 0.0
            cd_v[sl] = jnp.where(good, 1.0 / jnp.where(good, dtot, 1.0), 0.0)

        def _dr(b, _):
            r16 = cd_v[pl.ds(b * 16, 16)]
            for l in range(16):
                e = b * 16 + l
                rs = r16[l]
                for j in range(D // 16):
                    sj = pl.ds(j * 16, 16)
                    rows_b[0][e, sj] = rows_b[0][e, sj] * rs
            return 0
        lax.fori_loop(0, FB // 16, _dr, 0)
        pltpu.sync_copy(rows_b[0], out_hbm.at[pl.ds(row0, FB)])
        return 0
    lax.fori_loop(0, nch, _fin, 0)


def _sc_body(th0, th1, as0, at0, as1, at1, e0, e1, h0o, h1o,
             asrc_v, atgt_v, si0_v, ti0_v, si1_v, ti1_v, w0_v, w1_v,
             rows0_v, rows1_v, dtmp_v, cd_v, acc_sh, den_sp,
             si0s, si1s, sg0, sg1, ss0, ss1, sd0, sd1):
    c = lax.axis_index("c")
    si_b = (si0_v, si1_v)
    ti_b = (ti0_v, ti1_v)
    w_b = (w0_v, w1_v)
    rows_b = (rows0_v, rows1_v)
    sem_i = (si0s, si1s)
    sem_g = (sg0, sg1)
    sem_s = (ss0, ss1)
    sem_d = (sd0, sd1)

    @pl.when(c == 0)
    def _():
        _sc_work(th0, as0, at0, e0, h0o,
                 asrc_v, atgt_v, si_b, ti_b, w_b, rows_b,
                 dtmp_v, cd_v, acc_sh, den_sp, sem_i, sem_g, sem_s, sem_d)

    @pl.when(c == 1)
    def _():
        _sc_work(th1, as1, at1, e1, h1o,
                 asrc_v, atgt_v, si_b, ti_b, w_b, rows_b,
                 dtmp_v, cd_v, acc_sh, den_sp, sem_i, sem_g, sem_s, sem_d)


def _run_sc(th0, th1, as0, at0, as1, at1, e0, e1):
    f32 = jnp.float32
    i32 = jnp.int32
    mesh = plsc.VectorSubcoreMesh(core_axis_name="c", subcore_axis_name="s")
    call = pl.kernel(
        _sc_body,
        compiler_params=pltpu.CompilerParams(needs_layout_passes=False,
                                             use_tc_tiling_on_sc=False),
        out_type=[jax.ShapeDtypeStruct((N, D), f32),
                  jax.ShapeDtypeStruct((N, D), f32)],
        mesh=mesh,
        scratch_types=[
            pltpu.VMEM((N,), f32),
            pltpu.VMEM((N,), f32),
            pltpu.VMEM((CH,), i32),
            pltpu.VMEM((CH,), i32),
            pltpu.VMEM((CH,), i32),
            pltpu.VMEM((CH,), i32),
            pltpu.VMEM((CH,), f32),
            pltpu.VMEM((CH,), f32),
            pltpu.VMEM((CH, D), f32),
            pltpu.VMEM((CH, D), f32),
            pltpu.VMEM((FB,), f32),
            pltpu.VMEM((FB,), f32),
            pltpu.VMEM_SHARED((N, D), f32),
            pltpu.VMEM_SHARED((N,), f32),
            pltpu.SemaphoreType.DMA,
            pltpu.SemaphoreType.DMA,
            pltpu.SemaphoreType.DMA,
            pltpu.SemaphoreType.DMA,
            pltpu.SemaphoreType.DMA,
            pltpu.SemaphoreType.DMA,
            pltpu.SemaphoreType.DMA,
            pltpu.SemaphoreType.DMA,
        ],
    )
    return call(th0, th1, as0, at0, as1, at1, e0, e1)


def _ln(x, g, b):
    m = jnp.mean(x, axis=-1, keepdims=True)
    v = jnp.mean((x - m) * (x - m), axis=-1, keepdims=True)
    return (x - m) / jnp.sqrt(v + 1e-5) * g + b


def _epilogue_body(srch, h0n, h1n, g0b, g1b, semW, semb, prepW, prepb,
                   d0W, d0b, d0g, d0be, d1W, d1b, d1g, d1be,
                   rg, rbe, clsW, clsb, out_o):
    s = srch[...]
    h0 = h0n[...] + g0b[...]
    h1 = h1n[...] + g1b[...]
    sw = semW[...]
    sb = semb[...]
    a0 = s @ sw + sb
    a1 = h0 @ sw + sb
    a2 = h1 @ sw + sb
    att = jnp.concatenate([a0, a1, a2], axis=1)
    att = jnp.where(att > 0, att, 0.01 * att)
    att = att - jnp.max(att, axis=1, keepdims=True)
    ea = jnp.exp(att)
    p = ea / jnp.sum(ea, axis=1, keepdims=True)
    hp = p[:, 0:1] * s + p[:, 1:2] * h0 + p[:, 2:3] * h1
    h = hp @ prepW[...] + prepb[...]
    hs = h
    h = _ln(jnp.tanh(h @ d0W[...] + d0b[...]), d0g[...], d0be[...])
    h = _ln(jnp.tanh(h @ d1W[...] + d1b[...]), d1g[...], d1be[...])
    h = _ln(jnp.tanh(hs + h), rg[...], rbe[...])
    z = h @ clsW[...] + clsb[...]
    out_o[...] = 1.0 / (1.0 + jnp.exp(-z))


def _run_epilogue(srch, h0n, h1n, gat0_bias, gat1_bias, sem_W, sem_b,
                  prep_W, prep_b, dnn0_W, dnn0_b, dnn0_g, dnn0_be,
                  dnn1_W, dnn1_b, dnn1_g, dnn1_be, res_g, res_be,
                  cls_W, cls_b):
    row = pl.BlockSpec((RB, D), lambda i: (i, 0))
    mat = pl.BlockSpec((D, D), lambda i: (0, 0))
    vec = pl.BlockSpec((D,), lambda i: (0,))
    cvec = pl.BlockSpec((D, 1), lambda i: (0, 0))
    one = pl.BlockSpec((1,), lambda i: (0,))
    col = pl.BlockSpec((RB, 1), lambda i: (i, 0))
    return pl.pallas_call(
        _epilogue_body,
        grid=(NBLK,),
        in_specs=[row, row, row, vec, vec, cvec, one, mat, vec,
                  mat, vec, vec, vec, mat, vec, vec, vec,
                  vec, vec, cvec, one],
        out_specs=col,
        out_shape=jax.ShapeDtypeStruct((N, 1), jnp.float32),
    )(srch, h0n, h1n, gat0_bias, gat1_bias, sem_W, sem_b, prep_W, prep_b,
      dnn0_W, dnn0_b, dnn0_g, dnn0_be, dnn1_W, dnn1_b, dnn1_g, dnn1_be,
      res_g, res_be, cls_W, cls_b)


def kernel(src_feat, mp0_feat, mp1_feat, W_src, b_src, W_nbr, b_nbr,
           gat0_Wf, gat0_bf, gat0_Wa, gat0_ba, gat0_bias,
           gat1_Wf, gat1_bf, gat1_Wa, gat1_ba, gat1_bias,
           sem_W, sem_b, prep_W, prep_b,
           dnn0_W, dnn0_b, dnn0_g, dnn0_be,
           dnn1_W, dnn1_b, dnn1_g, dnn1_be,
           res_g, res_be, cls_W, cls_b,
           mp0_edge_index, mp1_edge_index):
    srch, th0, th1, as0, at0, as1, at1 = _run_prologue(
        src_feat, mp0_feat, mp1_feat, W_src, b_src, W_nbr, b_nbr,
        gat0_Wf, gat0_bf, gat0_Wa, gat0_ba, gat1_Wf, gat1_bf, gat1_Wa,
        gat1_ba)

    h0n, h1n = _run_sc(th0, th1,
                       as0.reshape(N), at0.reshape(N),
                       as1.reshape(N), at1.reshape(N),
                       mp0_edge_index, mp1_edge_index)

    out = _run_epilogue(srch, h0n, h1n, gat0_bias, gat1_bias, sem_W, sem_b,
                        prep_W, prep_b, dnn0_W, dnn0_b, dnn0_g, dnn0_be,
                        dnn1_W, dnn1_b, dnn1_g, dnn1_be, res_g, res_be,
                        cls_W, cls_b)
    return out.reshape(N)


# speedup vs baseline: 31.2407x; 1.2169x over previous
import functools

import jax
import jax.numpy as jnp
from jax import lax
from jax.experimental import pallas as pl
from jax.experimental.pallas import tpu as pltpu
from jax.experimental.pallas import tpu_sc as plsc

N = 10000
E = 320000
D = 128

RB = 1000
NBLK = N // RB

NTILE = 16
EPT = E // NTILE
CH = 80
NCHUNK = EPT // CH
FB = 80
NCHK = N // FB


def _prologue_body(src, m0, m1, Wsrc, bsrc, Wnbr, bnbr,
                   Wf0, bf0, Wa0, ba0, Wf1, bf1, Wa1, ba1,
                   srch_o, th0_o, th1_o, as0_o, at0_o, as1_o, at1_o):
    sh = src[...] @ Wsrc[...] + bsrc[...]
    srch_o[...] = sh
    f0 = m0[...] @ Wnbr[...] + bnbr[...]
    f1 = m1[...] @ Wnbr[...] + bnbr[...]
    t0 = f0 @ Wf0[...] + bf0[...]
    t1 = f1 @ Wf1[...] + bf1[...]
    th0_o[...] = t0
    th1_o[...] = t1
    wa0 = Wa0[...]
    wa1 = Wa1[...]
    v0 = Wf0[...] @ wa0[:D]
    v1 = Wf1[...] @ wa1[:D]
    c0 = bf0[...] @ wa0[:D] + ba0[...]
    c1 = bf1[...] @ wa1[:D] + ba1[...]
    as0_o[...] = sh @ v0 + c0
    at0_o[...] = t0 @ wa0[D:]
    as1_o[...] = sh @ v1 + c1
    at1_o[...] = t1 @ wa1[D:]


def _run_prologue(src_feat, mp0_feat, mp1_feat, W_src, b_src, W_nbr, b_nbr,
                  gat0_Wf, gat0_bf, gat0_Wa, gat0_ba,
                  gat1_Wf, gat1_bf, gat1_Wa, gat1_ba):
    row = pl.BlockSpec((RB, D), lambda i: (i, 0))
    mat = pl.BlockSpec((D, D), lambda i: (0, 0))
    vec = pl.BlockSpec((D,), lambda i: (0,))
    wa = pl.BlockSpec((2 * D, 1), lambda i: (0, 0))
    one = pl.BlockSpec((1,), lambda i: (0,))
    col = pl.BlockSpec((RB, 1), lambda i: (i, 0))
    f32 = jnp.float32
    return pl.pallas_call(
        _prologue_body,
        grid=(NBLK,),
        in_specs=[row, row, row, mat, vec, mat, vec,
                  mat, vec, wa, one, mat, vec, wa, one],
        out_specs=[row, row, row, col, col, col, col],
        out_shape=[jax.ShapeDtypeStruct((N, D), f32)] * 3 +
                  [jax.ShapeDtypeStruct((N, 1), f32)] * 4,
    )(src_feat, mp0_feat, mp1_feat, W_src, b_src, W_nbr, b_nbr,
      gat0_Wf, gat0_bf, gat0_Wa, gat0_ba, gat1_Wf, gat1_bf, gat1_Wa, gat1_ba)


def _sc_work(th_hbm, asrc_hbm, atgt_hbm, e_hbm, out_hbm,
             asrc_v, atgt_v, si_b, ti_b, ss_b, w_b, rows_b,
             dtmp_v, cd_v, acc_sh, den_sp,
             sem_i, sem_g, sem_s, sem_d):
    tec = lax.axis_index("s")
    ebase = tec * EPT

    nch = jnp.where(tec <= (NCHK - 1) % NTILE, NCHK // NTILE + 1,
                    NCHK // NTILE)

    def _zrow(e, _):
        for j in range(D // 16):
            rows_b[0][e, pl.ds(j * 16, 16)] = jnp.zeros((16,), jnp.float32)
        return 0
    lax.fori_loop(0, FB, _zrow, 0)
    for b in range(FB // 16):
        dtmp_v[pl.ds(b * 16, 16)] = jnp.zeros((16,), jnp.float32)

    def _zacc(k, _):
        c = tec + NTILE * k
        pltpu.sync_copy(rows_b[0], acc_sh.at[pl.ds(c * FB, FB)])
        pltpu.sync_copy(dtmp_v, den_sp.at[pl.ds(c * FB, FB)])
        return 0
    lax.fori_loop(0, nch, _zacc, 0)

    pltpu.sync_copy(asrc_hbm, asrc_v)
    pltpu.sync_copy(atgt_hbm, atgt_v)

    plsc.subcore_barrier()

    def _issue_idx(g, q):
        off = ebase + g * CH
        pltpu.async_copy(e_hbm.at[0, pl.ds(off, CH)], si_b[q], sem_i[q])
        pltpu.async_copy(e_hbm.at[1, pl.ds(off, CH)], ti_b[q], sem_i[q])

    def _wait_idx(g, q):
        off = ebase + g * CH
        pltpu.make_async_copy(e_hbm.at[0, pl.ds(off, CH)], si_b[q],
                              sem_i[q]).wait()
        pltpu.make_async_copy(e_hbm.at[1, pl.ds(off, CH)], ti_b[q],
                              sem_i[q]).wait()

    def _chunk(g, m, q, last):
        if q == 0:
            @pl.when(m >= 1)
            def _():
                pltpu.make_async_copy(w_b[0], den_sp.at[ss_b[0]],
                                      sem_d[0]).wait()
        else:
            @pl.when(m >= 1)
            def _():
                pltpu.make_async_copy(w_b[1], den_sp.at[ss_b[1]],
                                      sem_d[1]).wait()
        if not last:
            _issue_idx(g + 1, 1 - q)
        for b in range(CH // 16):
            sl = pl.ds(b * 16, 16)
            s16 = si_b[q][sl]
            t16 = ti_b[q][sl]
            x = (plsc.load_gather(asrc_v, [s16]) +
                 plsc.load_gather(atgt_v, [t16]))
            ex2 = jnp.exp(x + x)
            w_b[q][sl] = jnp.exp(1.0 - 2.0 / (ex2 + 1.0))
            ss_b[q][sl] = s16
        pltpu.async_copy(w_b[q], den_sp.at[ss_b[q]], sem_d[q], add=True)
        pltpu.make_async_copy(th_hbm.at[ti_b[q]], rows_b[q],
                              sem_g[q]).wait()
        if not last:
            if q == 0:
                @pl.when(m >= 1)
                def _():
                    pltpu.make_async_copy(rows_b[1], acc_sh.at[ss_b[1]],
                                          sem_s[1]).wait()
            else:
                pltpu.make_async_copy(rows_b[0], acc_sh.at[ss_b[0]],
                                      sem_s[0]).wait()
            _wait_idx(g + 1, 1 - q)
            pltpu.async_copy(th_hbm.at[ti_b[1 - q]], rows_b[1 - q],
                             sem_g[1 - q])

        def _scale(b, _):
            w16 = w_b[q][pl.ds(b * 16, 16)]
            for l in range(16):
                e = b * 16 + l
                ws = w16[l]
                for j in range(D // 16):
                    sj = pl.ds(j * 16, 16)
                    rows_b[q][e, sj] = rows_b[q][e, sj] * ws
            return 0
        lax.fori_loop(0, CH // 16, _scale, 0)

        pltpu.async_copy(rows_b[q], acc_sh.at[ss_b[q]], sem_s[q], add=True)

    _issue_idx(0, 0)
    _wait_idx(0, 0)
    pltpu.async_copy(th_hbm.at[ti_b[0]], rows_b[0], sem_g[0])

    def _pair(m, _):
        _chunk(2 * m, m, 0, False)

        @pl.when(m == NCHUNK // 2 - 1)
        def _():
            _chunk(2 * m + 1, m, 1, True)

        @pl.when(m < NCHUNK // 2 - 1)
        def _():
            _chunk(2 * m + 1, m, 1, False)
        return 0
    lax.fori_loop(0, NCHUNK // 2, _pair, 0)

    for q in (0, 1):
        pltpu.make_async_copy(rows_b[q], acc_sh.at[ss_b[q]], sem_s[q]).wait()
        pltpu.make_async_copy(w_b[q], den_sp.at[ss_b[q]], sem_d[q]).wait()

    plsc.subcore_barrier()


    def _fin(k, _):
        c = tec + NTILE * k
        row0 = c * FB
        pltpu.sync_copy(acc_sh.at[pl.ds(row0, FB)], rows_b[0])
        pltpu.sync_copy(den_sp.at[pl.ds(row0, FB)], dtmp_v)
        for b in range(FB // 16):
            sl = pl.ds(b * 16, 16)
            dtot = dtmp_v[sl]
            good = dtot > 0.0
            cd_v[sl] = jnp.where(good, 1.0 / jnp.where(good, dtot, 1.0), 0.0)

        def _dr(b, _):
            r16 = cd_v[pl.ds(b * 16, 16)]
            for l in range(16):
                e = b * 16 + l
                rs = r16[l]
                for j in range(D // 16):
                    sj = pl.ds(j * 16, 16)
                    rows_b[0][e, sj] = rows_b[0][e, sj] * rs
            return 0
        lax.fori_loop(0, FB // 16, _dr, 0)
        pltpu.sync_copy(rows_b[0], out_hbm.at[pl.ds(row0, FB)])
        return 0
    lax.fori_loop(0, nch, _fin, 0)


def _sc_body(th0, th1, as0, at0, as1, at1, e0, e1, h0o, h1o,
             asrc_v, atgt_v, si0_v, ti0_v, si1_v, ti1_v, ss0_v, ss1_v,
             w0_v, w1_v,
             rows0_v, rows1_v, dtmp_v, cd_v, acc_sh, den_sp,
             si0s, si1s, sg0, sg1, ss0, ss1, sd0, sd1):
    c = lax.axis_index("c")
    si_b = (si0_v, si1_v)
    ti_b = (ti0_v, ti1_v)
    ss_b = (ss0_v, ss1_v)
    w_b = (w0_v, w1_v)
    rows_b = (rows0_v, rows1_v)
    sem_i = (si0s, si1s)
    sem_g = (sg0, sg1)
    sem_s = (ss0, ss1)
    sem_d = (sd0, sd1)

    @pl.when(c == 0)
    def _():
        _sc_work(th0, as0, at0, e0, h0o,
                 asrc_v, atgt_v, si_b, ti_b, ss_b, w_b, rows_b,
                 dtmp_v, cd_v, acc_sh, den_sp, sem_i, sem_g, sem_s, sem_d)

    @pl.when(c == 1)
    def _():
        _sc_work(th1, as1, at1, e1, h1o,
                 asrc_v, atgt_v, si_b, ti_b, ss_b, w_b, rows_b,
                 dtmp_v, cd_v, acc_sh, den_sp, sem_i, sem_g, sem_s, sem_d)


def _run_sc(th0, th1, as0, at0, as1, at1, e0, e1):
    f32 = jnp.float32
    i32 = jnp.int32
    mesh = plsc.VectorSubcoreMesh(core_axis_name="c", subcore_axis_name="s")
    call = pl.kernel(
        _sc_body,
        compiler_params=pltpu.CompilerParams(needs_layout_passes=False,
                                             use_tc_tiling_on_sc=False),
        out_type=[jax.ShapeDtypeStruct((N, D), f32),
                  jax.ShapeDtypeStruct((N, D), f32)],
        mesh=mesh,
        scratch_types=[
            pltpu.VMEM((N,), f32),
            pltpu.VMEM((N,), f32),
            pltpu.VMEM((CH,), i32),
            pltpu.VMEM((CH,), i32),
            pltpu.VMEM((CH,), i32),
            pltpu.VMEM((CH,), i32),
            pltpu.VMEM((CH,), i32),
            pltpu.VMEM((CH,), i32),
            pltpu.VMEM((CH,), f32),
            pltpu.VMEM((CH,), f32),
            pltpu.VMEM((CH, D), f32),
            pltpu.VMEM((CH, D), f32),
            pltpu.VMEM((FB,), f32),
            pltpu.VMEM((FB,), f32),
            pltpu.VMEM_SHARED((N, D), f32),
            pltpu.VMEM_SHARED((N,), f32),
            pltpu.SemaphoreType.DMA,
            pltpu.SemaphoreType.DMA,
            pltpu.SemaphoreType.DMA,
            pltpu.SemaphoreType.DMA,
            pltpu.SemaphoreType.DMA,
            pltpu.SemaphoreType.DMA,
            pltpu.SemaphoreType.DMA,
            pltpu.SemaphoreType.DMA,
        ],
    )
    return call(th0, th1, as0, at0, as1, at1, e0, e1)


def _ln(x, g, b):
    m = jnp.mean(x, axis=-1, keepdims=True)
    v = jnp.mean((x - m) * (x - m), axis=-1, keepdims=True)
    return (x - m) / jnp.sqrt(v + 1e-5) * g + b


def _epilogue_body(srch, h0n, h1n, g0b, g1b, semW, semb, prepW, prepb,
                   d0W, d0b, d0g, d0be, d1W, d1b, d1g, d1be,
                   rg, rbe, clsW, clsb, out_o):
    s = srch[...]
    h0 = h0n[...] + g0b[...]
    h1 = h1n[...] + g1b[...]
    sw = semW[...]
    sb = semb[...]
    a0 = s @ sw + sb
    a1 = h0 @ sw + sb
    a2 = h1 @ sw + sb
    att = jnp.concatenate([a0, a1, a2], axis=1)
    att = jnp.where(att > 0, att, 0.01 * att)
    att = att - jnp.max(att, axis=1, keepdims=True)
    ea = jnp.exp(att)
    p = ea / jnp.sum(ea, axis=1, keepdims=True)
    hp = p[:, 0:1] * s + p[:, 1:2] * h0 + p[:, 2:3] * h1
    h = hp @ prepW[...] + prepb[...]
    hs = h
    h = _ln(jnp.tanh(h @ d0W[...] + d0b[...]), d0g[...], d0be[...])
    h = _ln(jnp.tanh(h @ d1W[...] + d1b[...]), d1g[...], d1be[...])
    h = _ln(jnp.tanh(hs + h), rg[...], rbe[...])
    z = h @ clsW[...] + clsb[...]
    out_o[...] = 1.0 / (1.0 + jnp.exp(-z))


def _run_epilogue(srch, h0n, h1n, gat0_bias, gat1_bias, sem_W, sem_b,
                  prep_W, prep_b, dnn0_W, dnn0_b, dnn0_g, dnn0_be,
                  dnn1_W, dnn1_b, dnn1_g, dnn1_be, res_g, res_be,
                  cls_W, cls_b):
    row = pl.BlockSpec((RB, D), lambda i: (i, 0))
    mat = pl.BlockSpec((D, D), lambda i: (0, 0))
    vec = pl.BlockSpec((D,), lambda i: (0,))
    cvec = pl.BlockSpec((D, 1), lambda i: (0, 0))
    one = pl.BlockSpec((1,), lambda i: (0,))
    col = pl.BlockSpec((RB, 1), lambda i: (i, 0))
    return pl.pallas_call(
        _epilogue_body,
        grid=(NBLK,),
        in_specs=[row, row, row, vec, vec, cvec, one, mat, vec,
                  mat, vec, vec, vec, mat, vec, vec, vec,
                  vec, vec, cvec, one],
        out_specs=col,
        out_shape=jax.ShapeDtypeStruct((N, 1), jnp.float32),
    )(srch, h0n, h1n, gat0_bias, gat1_bias, sem_W, sem_b, prep_W, prep_b,
      dnn0_W, dnn0_b, dnn0_g, dnn0_be, dnn1_W, dnn1_b, dnn1_g, dnn1_be,
      res_g, res_be, cls_W, cls_b)


def kernel(src_feat, mp0_feat, mp1_feat, W_src, b_src, W_nbr, b_nbr,
           gat0_Wf, gat0_bf, gat0_Wa, gat0_ba, gat0_bias,
           gat1_Wf, gat1_bf, gat1_Wa, gat1_ba, gat1_bias,
           sem_W, sem_b, prep_W, prep_b,
           dnn0_W, dnn0_b, dnn0_g, dnn0_be,
           dnn1_W, dnn1_b, dnn1_g, dnn1_be,
           res_g, res_be, cls_W, cls_b,
           mp0_edge_index, mp1_edge_index):
    srch, th0, th1, as0, at0, as1, at1 = _run_prologue(
        src_feat, mp0_feat, mp1_feat, W_src, b_src, W_nbr, b_nbr,
        gat0_Wf, gat0_bf, gat0_Wa, gat0_ba, gat1_Wf, gat1_bf, gat1_Wa,
        gat1_ba)

    h0n, h1n = _run_sc(th0, th1,
                       as0.reshape(N), at0.reshape(N),
                       as1.reshape(N), at1.reshape(N),
                       mp0_edge_index, mp1_edge_index)

    out = _run_epilogue(srch, h0n, h1n, gat0_bias, gat1_bias, sem_W, sem_b,
                        prep_W, prep_b, dnn0_W, dnn0_b, dnn0_g, dnn0_be,
                        dnn1_W, dnn1_b, dnn1_g, dnn1_be, res_g, res_be,
                        cls_W, cls_b)
    return out.reshape(N)
